# baseline (speedup 1.0000x reference)
import functools

import jax
import jax.numpy as jnp
from jax import lax
from jax.experimental import pallas as pl
from jax.experimental.pallas import tpu as pltpu
from jax.experimental.pallas import tpu_sc as plsc

N_USERS = 25000
N_ITEMS = 25000
N_NODES = N_USERS + N_ITEMS
DIM = 64
BATCH = 4096
HALF = N_NODES // 2
NC, NS, L = 2, 16, 16
EB = 128
TILE_ROWS = 1568
SLAB_ROWS = TILE_ROWS * NS
DUMMY = HALF
CHUNK = 112
NCHUNK = TILE_ROWS // CHUNK
PB = BATCH // (NC * NS)

_mesh = plsc.VectorSubcoreMesh(
    core_axis_name="c", subcore_axis_name="s", num_cores=NC, num_subcores=NS
)


def _layer_body(emb_in, src_p, dst_p, w_p, meta, emb_out,
                slab, obuf, rows0, rows1, sbuf0, sbuf1, dbuf0, dbuf1,
                wbuf0, wbuf1, dl0, dl1, mrow, gsem, ssem, esem):
    cid = lax.axis_index("c")
    sid = lax.axis_index("s")
    iot = lax.iota(jnp.int32, L)
    zero16 = jnp.zeros((L,), jnp.float32)

    def zrow(r, _):
        for q in range(DIM // L):
            obuf[r, pl.ds(q * L, L)] = zero16
        return 0
    lax.fori_loop(0, CHUNK, zrow, 0)
    row0 = sid * TILE_ROWS
    for k in range(NCHUNK):
        pltpu.sync_copy(obuf, slab.at[pl.ds(row0 + k * CHUNK, CHUNK)])
    plsc.subcore_barrier()

    pltpu.sync_copy(meta.at[cid], mrow)
    sel = iot == sid
    def pick(off):
        return jnp.sum(jnp.where(sel, mrow[pl.ds(off, L)], 0))
    start = pick(0)
    nblocks = pick(L)
    end = pick(2 * L)
    dlo = cid * HALF

    BUFS = ((rows0, sbuf0, dbuf0, wbuf0, dl0),
            (rows1, sbuf1, dbuf1, wbuf1, dl1))

    def start_edges(i, bufs):
        _, sb, db, wb, _ = bufs
        e0 = pl.multiple_of(start + i * EB, 8)
        pltpu.async_copy(src_p.at[pl.ds(e0, EB)], sb, esem)
        pltpu.async_copy(dst_p.at[pl.ds(e0, EB)], db, esem)
        pltpu.async_copy(w_p.at[pl.ds(e0, EB)], wb, esem)

    def wait_edges(bufs):
        _, sb, db, wb, _ = bufs
        pltpu.make_async_copy(src_p.at[pl.ds(0, EB)], sb, esem).wait()
        pltpu.make_async_copy(dst_p.at[pl.ds(0, EB)], db, esem).wait()
        pltpu.make_async_copy(w_p.at[pl.ds(0, EB)], wb, esem).wait()

    def mask_and_gather(i, bufs):
        rw, sb, db, _, dl = bufs
        e0 = start + i * EB
        for k in range(EB // L):
            dv = db[pl.ds(k * L, L)]
            ev = e0 + k * L + iot
            ok = (ev < end) & (dv >= dlo) & (dv < dlo + HALF)
            dl[pl.ds(k * L, L)] = jnp.where(ok, dv - dlo, DUMMY)
        pltpu.async_copy(emb_in.at[sb], rw, gsem)

    def wait_gather(bufs):
        pltpu.make_async_copy(emb_in.at[bufs[1]], bufs[0], gsem).wait()

    def scale(bufs):
        rw, _, _, wb, _ = bufs

        def srow(g, _):
            r = g * L
            wv = wb[pl.ds(r, L)]
            for u in range(L):
                wu = jnp.full((L,), wv[u])
                for q in range(DIM // L):
                    s = pl.ds(q * L, L)
                    rw[r + u, s] = rw[r + u, s] * wu
            return 0
        lax.fori_loop(0, EB // L, srow, 0)

    def start_scatter(bufs):
        pltpu.async_copy(bufs[0], slab.at[bufs[4]], ssem, add=True)

    def wait_scatter(bufs):
        pltpu.make_async_copy(bufs[0], slab.at[bufs[4]], ssem).wait()

    @pl.when(nblocks > 0)
    def _():
        start_edges(0, BUFS[0])
        wait_edges(BUFS[0])
        mask_and_gather(0, BUFS[0])

    def step(i, cur, nxt):
        @pl.when(i < nblocks)
        def _():
            nok = (i + 1) < nblocks

            @pl.when(nok)
            def _():
                start_edges(i + 1, nxt)

            wait_gather(cur)

            @pl.when(i >= 1)
            def _():
                wait_scatter(nxt)

            @pl.when(nok)
            def _():
                wait_edges(nxt)
                mask_and_gather(i + 1, nxt)

            scale(cur)
            start_scatter(cur)

    def pair(i2, _):
        i = i2 * 2
        step(i, BUFS[0], BUFS[1])
        step(i + 1, BUFS[1], BUFS[0])
        return 0

    lax.fori_loop(0, (nblocks + 1) // 2, pair, 0)

    @pl.when(nblocks % 2 == 1)
    def _():
        wait_scatter(BUFS[0])

    @pl.when((nblocks > 0) & (nblocks % 2 == 0))
    def _():
        wait_scatter(BUFS[1])

    plsc.subcore_barrier()

    gbase = cid * HALF + row0

    def copy_chunk(off, n):
        pltpu.sync_copy(slab.at[pl.ds(row0 + off, n)], obuf.at[pl.ds(0, n)])
        pltpu.sync_copy(obuf.at[pl.ds(0, n)], emb_out.at[pl.ds(gbase + off, n)])

    for k in range(NCHUNK - 1):
        copy_chunk(k * CHUNK, CHUNK)
    tail_off = (NCHUNK - 1) * CHUNK

    @pl.when(sid < NS - 1)
    def _():
        copy_chunk(tail_off, CHUNK)

    @pl.when(sid == NS - 1)
    def _():
        copy_chunk(tail_off, HALF - (NS - 1) * TILE_ROWS - tail_off)


_layer = functools.partial(
    pl.kernel,
    out_type=jax.ShapeDtypeStruct((N_NODES, DIM), jnp.float32),
    mesh=_mesh,
    compiler_params=pltpu.CompilerParams(needs_layout_passes=False, use_tc_tiling_on_sc=False),
    scratch_types=[
        pltpu.VMEM_SHARED((SLAB_ROWS, DIM), jnp.float32),
        pltpu.VMEM((CHUNK, DIM), jnp.float32),
        pltpu.VMEM((EB, DIM), jnp.float32),
        pltpu.VMEM((EB, DIM), jnp.float32),
        pltpu.VMEM((EB,), jnp.int32),
        pltpu.VMEM((EB,), jnp.int32),
        pltpu.VMEM((EB,), jnp.int32),
        pltpu.VMEM((EB,), jnp.int32),
        pltpu.VMEM((EB,), jnp.float32),
        pltpu.VMEM((EB,), jnp.float32),
        pltpu.VMEM((EB,), jnp.int32),
        pltpu.VMEM((EB,), jnp.int32),
        pltpu.VMEM((3 * L,), jnp.int32),
        pltpu.SemaphoreType.DMA,
        pltpu.SemaphoreType.DMA,
        pltpu.SemaphoreType.DMA,
    ],
)(_layer_body)


def _final_body(uidx_hbm, iidx_hbm, t0, t1, t2, t3, gamma,
                uidxb, iidxb, usum, isum, tmp, obuf1, gsem, hsem):
    cid = lax.axis_index("c")
    sid = lax.axis_index("s")
    iot = lax.iota(jnp.int32, L)
    wid = cid * NS + sid
    pbase = pl.multiple_of(wid * PB, 8)
    pltpu.sync_copy(uidx_hbm.at[pl.ds(pbase, PB)], uidxb)
    pltpu.sync_copy(iidx_hbm.at[pl.ds(pbase, PB)], iidxb)

    gu = pltpu.async_copy(t0.at[uidxb], usum, gsem)
    gi = pltpu.async_copy(t0.at[iidxb], isum, hsem)
    gu.wait()
    gi.wait()

    def accum(dst):
        def addrow(r, _):
            for q in range(DIM // L):
                s = pl.ds(q * L, L)
                dst[r, s] = dst[r, s] + tmp[r, s]
            return 0
        lax.fori_loop(0, PB, addrow, 0)

    for tk in (t1, t2, t3):
        pltpu.async_copy(tk.at[uidxb], tmp, gsem).wait()
        accum(usum)
        pltpu.async_copy(tk.at[iidxb], tmp, hsem).wait()
        accum(isum)

    def grp(g, _):
        ridx = g * L + iot

        def dstep(d, acc):
            colv = jnp.full((L,), d, jnp.int32)
            u = plsc.load_gather(usum, [ridx, colv])
            v = plsc.load_gather(isum, [ridx, colv])
            return acc + u * v

        acc = lax.fori_loop(0, DIM, dstep, jnp.zeros((L,), jnp.float32))
        obuf1[pl.ds(g * L, L)] = acc * (1.0 / 16.0)
        return 0

    lax.fori_loop(0, PB // L, grp, 0)
    pltpu.sync_copy(obuf1, gamma.at[pl.ds(pbase, PB)])


_final = functools.partial(
    pl.kernel,
    out_type=jax.ShapeDtypeStruct((BATCH,), jnp.float32),
    mesh=_mesh,
    compiler_params=pltpu.CompilerParams(needs_layout_passes=False, use_tc_tiling_on_sc=False),
    scratch_types=[
        pltpu.VMEM((PB,), jnp.int32),
        pltpu.VMEM((PB,), jnp.int32),
        pltpu.VMEM((PB, DIM), jnp.float32),
        pltpu.VMEM((PB, DIM), jnp.float32),
        pltpu.VMEM((PB, DIM), jnp.float32),
        pltpu.VMEM((PB,), jnp.float32),
        pltpu.SemaphoreType.DMA,
        pltpu.SemaphoreType.DMA,
    ],
)(_final_body)


def _build_meta(m, n_edges):
    e_total = jnp.int32(n_edges)
    t = jnp.arange(NS, dtype=jnp.int32)
    c0 = ((m + NS - 1) // NS + 7) // 8 * 8
    s0 = t * c0
    e0 = jnp.minimum(s0 + c0, m)
    nb0 = jnp.maximum((e0 - s0 + EB - 1) // EB, 0)
    m8 = (m // 8) * 8
    c1 = ((e_total - m8 + NS - 1) // NS + 7) // 8 * 8
    s1 = m8 + t * c1
    e1 = jnp.minimum(s1 + c1, e_total)
    nb1 = jnp.maximum((e1 - s1 + EB - 1) // EB, 0)
    return jnp.stack([
        jnp.concatenate([s0, nb0, e0]),
        jnp.concatenate([s1, nb1, e1]),
    ])


def kernel(users, items, edge_src, edge_dst, edge_weight, user_emb, item_emb):
    users = users.astype(jnp.int32)
    items = items.astype(jnp.int32) + N_USERS
    edge_src = edge_src.astype(jnp.int32)
    edge_dst = edge_dst.astype(jnp.int32)
    emb0 = jnp.concatenate([user_emb, item_emb], axis=0)
    m = jnp.searchsorted(edge_dst, HALF).astype(jnp.int32)
    meta = _build_meta(m, edge_src.shape[0])
    src_p = jnp.concatenate([edge_src, jnp.zeros((EB,), jnp.int32)])
    dst_p = jnp.concatenate([edge_dst, jnp.full((EB,), N_NODES, jnp.int32)])
    w_p = jnp.concatenate([edge_weight, jnp.zeros((EB,), jnp.float32)])
    e1 = _layer(emb0, src_p, dst_p, w_p, meta)
    e2 = _layer(e1, src_p, dst_p, w_p, meta)
    e3 = _layer(e2, src_p, dst_p, w_p, meta)
    return _final(users, items, emb0, e1, e2, e3)


# speedup vs baseline: 2.1954x; 2.1954x over previous
import functools

import jax
import jax.numpy as jnp
from jax import lax
from jax.experimental import pallas as pl
from jax.experimental.pallas import tpu as pltpu
from jax.experimental.pallas import tpu_sc as plsc

N_USERS = 25000
N_ITEMS = 25000
N_NODES = N_USERS + N_ITEMS
DIM = 64
BATCH = 4096
HALF = N_NODES // 2
NC, NS, L = 2, 16, 16
EB = 128
TILE_ROWS = 1568
SLAB_ROWS = TILE_ROWS * NS
DUMMY = HALF
CHUNK = 112
NCHUNK = TILE_ROWS // CHUNK
PB = BATCH // (NC * NS)

_mesh = plsc.VectorSubcoreMesh(
    core_axis_name="c", subcore_axis_name="s", num_cores=NC, num_subcores=NS
)


def _layer_body(emb_in, src_p, dst_p, w_p, meta, emb_out,
                slab, obuf, rows0, rows1, sbuf0, sbuf1, dbuf0, dbuf1,
                wbuf0, wbuf1, dl0, dl1, mrow, gsem, ssem, esem):
    cid = lax.axis_index("c")
    sid = lax.axis_index("s")
    iot = lax.iota(jnp.int32, L)
    zero16 = jnp.zeros((L,), jnp.float32)

    def zrow(r, _):
        for q in range(DIM // L):
            obuf[r, pl.ds(q * L, L)] = zero16
        return 0
    lax.fori_loop(0, CHUNK, zrow, 0)
    row0 = sid * TILE_ROWS
    for k in range(NCHUNK):
        pltpu.sync_copy(obuf, slab.at[pl.ds(row0 + k * CHUNK, CHUNK)])
    plsc.subcore_barrier()

    pltpu.sync_copy(meta.at[cid], mrow)
    sel = iot == sid
    def pick(off):
        return jnp.sum(jnp.where(sel, mrow[pl.ds(off, L)], 0))
    start = pick(0)
    nblocks = pick(L)
    end = pick(2 * L)
    dlo = cid * HALF

    BUFS = ((rows0, sbuf0, dbuf0, wbuf0, dl0),
            (rows1, sbuf1, dbuf1, wbuf1, dl1))

    def start_edges(i, bufs):
        _, sb, db, wb, _ = bufs
        e0 = pl.multiple_of(start + i * EB, 8)
        pltpu.async_copy(src_p.at[pl.ds(e0, EB)], sb, esem)
        pltpu.async_copy(dst_p.at[pl.ds(e0, EB)], db, esem)
        pltpu.async_copy(w_p.at[pl.ds(e0, EB)], wb, esem)

    def wait_edges(bufs):
        _, sb, db, wb, _ = bufs
        pltpu.make_async_copy(src_p.at[pl.ds(0, EB)], sb, esem).wait()
        pltpu.make_async_copy(dst_p.at[pl.ds(0, EB)], db, esem).wait()
        pltpu.make_async_copy(w_p.at[pl.ds(0, EB)], wb, esem).wait()

    def mask_and_gather(i, bufs):
        rw, sb, db, _, dl = bufs
        e0 = start + i * EB
        for k in range(EB // L):
            dv = db[pl.ds(k * L, L)]
            ev = e0 + k * L + iot
            ok = (ev < end) & (dv >= dlo) & (dv < dlo + HALF)
            dl[pl.ds(k * L, L)] = jnp.where(ok, dv - dlo, DUMMY)
        pltpu.async_copy(emb_in.at[sb], rw, gsem)

    def wait_gather(bufs):
        pltpu.make_async_copy(emb_in.at[bufs[1]], bufs[0], gsem).wait()

    def scale(bufs):
        rw, _, _, wb, _ = bufs

        @plsc.parallel_loop(0, EB, 4, unroll=2)
        def _(r):
            ws = [plsc.load_gather(wb, [jnp.full((L,), r + u, jnp.int32)])
                  for u in range(4)]
            for q in range(DIM // L):
                for u in range(4):
                    s = pl.ds(q * L, L)
                    rw[r + u, s] = rw[r + u, s] * ws[u]

    def start_scatter(bufs):
        pltpu.async_copy(bufs[0], slab.at[bufs[4]], ssem, add=True)

    def wait_scatter(bufs):
        pltpu.make_async_copy(bufs[0], slab.at[bufs[4]], ssem).wait()

    @pl.when(nblocks > 0)
    def _():
        start_edges(0, BUFS[0])
        wait_edges(BUFS[0])
        mask_and_gather(0, BUFS[0])

    def step(i, cur, nxt):
        @pl.when(i < nblocks)
        def _():
            nok = (i + 1) < nblocks

            @pl.when(nok)
            def _():
                start_edges(i + 1, nxt)

            wait_gather(cur)

            @pl.when(i >= 1)
            def _():
                wait_scatter(nxt)

            @pl.when(nok)
            def _():
                wait_edges(nxt)
                mask_and_gather(i + 1, nxt)

            scale(cur)
            start_scatter(cur)

    def pair(i2, _):
        i = i2 * 2
        step(i, BUFS[0], BUFS[1])
        step(i + 1, BUFS[1], BUFS[0])
        return 0

    lax.fori_loop(0, (nblocks + 1) // 2, pair, 0)

    @pl.when(nblocks % 2 == 1)
    def _():
        wait_scatter(BUFS[0])

    @pl.when((nblocks > 0) & (nblocks % 2 == 0))
    def _():
        wait_scatter(BUFS[1])

    plsc.subcore_barrier()

    gbase = cid * HALF + row0

    def copy_chunk(off, n):
        pltpu.sync_copy(slab.at[pl.ds(row0 + off, n)], obuf.at[pl.ds(0, n)])
        pltpu.sync_copy(obuf.at[pl.ds(0, n)], emb_out.at[pl.ds(gbase + off, n)])

    for k in range(NCHUNK - 1):
        copy_chunk(k * CHUNK, CHUNK)
    tail_off = (NCHUNK - 1) * CHUNK

    @pl.when(sid < NS - 1)
    def _():
        copy_chunk(tail_off, CHUNK)

    @pl.when(sid == NS - 1)
    def _():
        copy_chunk(tail_off, HALF - (NS - 1) * TILE_ROWS - tail_off)


_layer = functools.partial(
    pl.kernel,
    out_type=jax.ShapeDtypeStruct((N_NODES, DIM), jnp.float32),
    mesh=_mesh,
    compiler_params=pltpu.CompilerParams(needs_layout_passes=False, use_tc_tiling_on_sc=False),
    scratch_types=[
        pltpu.VMEM_SHARED((SLAB_ROWS, DIM), jnp.float32),
        pltpu.VMEM((CHUNK, DIM), jnp.float32),
        pltpu.VMEM((EB, DIM), jnp.float32),
        pltpu.VMEM((EB, DIM), jnp.float32),
        pltpu.VMEM((EB,), jnp.int32),
        pltpu.VMEM((EB,), jnp.int32),
        pltpu.VMEM((EB,), jnp.int32),
        pltpu.VMEM((EB,), jnp.int32),
        pltpu.VMEM((EB,), jnp.float32),
        pltpu.VMEM((EB,), jnp.float32),
        pltpu.VMEM((EB,), jnp.int32),
        pltpu.VMEM((EB,), jnp.int32),
        pltpu.VMEM((3 * L,), jnp.int32),
        pltpu.SemaphoreType.DMA,
        pltpu.SemaphoreType.DMA,
        pltpu.SemaphoreType.DMA,
    ],
)(_layer_body)


def _final_body(uidx_hbm, iidx_hbm, t0, t1, t2, t3, gamma,
                uidxb, iidxb, usum, isum, tmp, obuf1, gsem, hsem):
    cid = lax.axis_index("c")
    sid = lax.axis_index("s")
    iot = lax.iota(jnp.int32, L)
    wid = cid * NS + sid
    pbase = pl.multiple_of(wid * PB, 8)
    pltpu.sync_copy(uidx_hbm.at[pl.ds(pbase, PB)], uidxb)
    pltpu.sync_copy(iidx_hbm.at[pl.ds(pbase, PB)], iidxb)

    gu = pltpu.async_copy(t0.at[uidxb], usum, gsem)
    gi = pltpu.async_copy(t0.at[iidxb], isum, hsem)
    gu.wait()
    gi.wait()

    def accum(dst):
        def addrow(r, _):
            for q in range(DIM // L):
                s = pl.ds(q * L, L)
                dst[r, s] = dst[r, s] + tmp[r, s]
            return 0
        lax.fori_loop(0, PB, addrow, 0)

    for tk in (t1, t2, t3):
        pltpu.async_copy(tk.at[uidxb], tmp, gsem).wait()
        accum(usum)
        pltpu.async_copy(tk.at[iidxb], tmp, hsem).wait()
        accum(isum)

    def grp(g, _):
        ridx = g * L + iot

        def dstep(d, acc):
            colv = jnp.full((L,), d, jnp.int32)
            u = plsc.load_gather(usum, [ridx, colv])
            v = plsc.load_gather(isum, [ridx, colv])
            return acc + u * v

        acc = lax.fori_loop(0, DIM, dstep, jnp.zeros((L,), jnp.float32))
        obuf1[pl.ds(g * L, L)] = acc * (1.0 / 16.0)
        return 0

    lax.fori_loop(0, PB // L, grp, 0)
    pltpu.sync_copy(obuf1, gamma.at[pl.ds(pbase, PB)])


_final = functools.partial(
    pl.kernel,
    out_type=jax.ShapeDtypeStruct((BATCH,), jnp.float32),
    mesh=_mesh,
    compiler_params=pltpu.CompilerParams(needs_layout_passes=False, use_tc_tiling_on_sc=False),
    scratch_types=[
        pltpu.VMEM((PB,), jnp.int32),
        pltpu.VMEM((PB,), jnp.int32),
        pltpu.VMEM((PB, DIM), jnp.float32),
        pltpu.VMEM((PB, DIM), jnp.float32),
        pltpu.VMEM((PB, DIM), jnp.float32),
        pltpu.VMEM((PB,), jnp.float32),
        pltpu.SemaphoreType.DMA,
        pltpu.SemaphoreType.DMA,
    ],
)(_final_body)


def _build_meta(m, n_edges):
    e_total = jnp.int32(n_edges)
    t = jnp.arange(NS, dtype=jnp.int32)
    c0 = ((m + NS - 1) // NS + 7) // 8 * 8
    s0 = t * c0
    e0 = jnp.minimum(s0 + c0, m)
    nb0 = jnp.maximum((e0 - s0 + EB - 1) // EB, 0)
    m8 = (m // 8) * 8
    c1 = ((e_total - m8 + NS - 1) // NS + 7) // 8 * 8
    s1 = m8 + t * c1
    e1 = jnp.minimum(s1 + c1, e_total)
    nb1 = jnp.maximum((e1 - s1 + EB - 1) // EB, 0)
    return jnp.stack([
        jnp.concatenate([s0, nb0, e0]),
        jnp.concatenate([s1, nb1, e1]),
    ])


def kernel(users, items, edge_src, edge_dst, edge_weight, user_emb, item_emb):
    users = users.astype(jnp.int32)
    items = items.astype(jnp.int32) + N_USERS
    edge_src = edge_src.astype(jnp.int32)
    edge_dst = edge_dst.astype(jnp.int32)
    emb0 = jnp.concatenate([user_emb, item_emb], axis=0)
    m = jnp.searchsorted(edge_dst, HALF).astype(jnp.int32)
    meta = _build_meta(m, edge_src.shape[0])
    src_p = jnp.concatenate([edge_src, jnp.zeros((EB,), jnp.int32)])
    dst_p = jnp.concatenate([edge_dst, jnp.full((EB,), N_NODES, jnp.int32)])
    w_p = jnp.concatenate([edge_weight, jnp.zeros((EB,), jnp.float32)])
    e1 = _layer(emb0, src_p, dst_p, w_p, meta)
    e2 = _layer(e1, src_p, dst_p, w_p, meta)
    e3 = _layer(e2, src_p, dst_p, w_p, meta)
    return _final(users, items, emb0, e1, e2, e3)


# speedup vs baseline: 2.2844x; 1.0405x over previous
import functools

import jax
import jax.numpy as jnp
from jax import lax
from jax.experimental import pallas as pl
from jax.experimental.pallas import tpu as pltpu
from jax.experimental.pallas import tpu_sc as plsc

N_USERS = 25000
N_ITEMS = 25000
N_NODES = N_USERS + N_ITEMS
DIM = 64
BATCH = 4096
HALF = N_NODES // 2
NC, NS, L = 2, 16, 16
EB = 128
TILE_ROWS = 1568
SLAB_ROWS = TILE_ROWS * NS
DUMMY = HALF
CHUNK = 112
NCHUNK = TILE_ROWS // CHUNK
PB = BATCH // (NC * NS)

_mesh = plsc.VectorSubcoreMesh(
    core_axis_name="c", subcore_axis_name="s", num_cores=NC, num_subcores=NS
)


def _layer_body(emb_in, src_p, dst_p, w_p, meta, emb_out,
                slab, obuf, rows0, rows1, sbuf0, sbuf1, dbuf0, dbuf1,
                wbuf0, wbuf1, dl0, dl1, mrow, gsem, ssem, esem, gsem2):
    cid = lax.axis_index("c")
    sid = lax.axis_index("s")
    iot = lax.iota(jnp.int32, L)
    zero16 = jnp.zeros((L,), jnp.float32)

    def zrow(r, _):
        for q in range(DIM // L):
            obuf[r, pl.ds(q * L, L)] = zero16
        return 0
    lax.fori_loop(0, CHUNK, zrow, 0)
    row0 = sid * TILE_ROWS
    for k in range(NCHUNK):
        pltpu.sync_copy(obuf, slab.at[pl.ds(row0 + k * CHUNK, CHUNK)])
    plsc.subcore_barrier()

    pltpu.sync_copy(meta.at[cid], mrow)
    sel = iot == sid
    def pick(off):
        return jnp.sum(jnp.where(sel, mrow[pl.ds(off, L)], 0))
    start = pick(0)
    nblocks = pick(L)
    end = pick(2 * L)
    dlo = cid * HALF

    BUFS = ((rows0, sbuf0, dbuf0, wbuf0, dl0),
            (rows1, sbuf1, dbuf1, wbuf1, dl1))

    def start_edges(i, bufs):
        _, sb, db, wb, _ = bufs
        e0 = pl.multiple_of(start + i * EB, 8)
        pltpu.async_copy(src_p.at[pl.ds(e0, EB)], sb, esem)
        pltpu.async_copy(dst_p.at[pl.ds(e0, EB)], db, esem)
        pltpu.async_copy(w_p.at[pl.ds(e0, EB)], wb, esem)

    def wait_edges(bufs):
        _, sb, db, wb, _ = bufs
        pltpu.make_async_copy(src_p.at[pl.ds(0, EB)], sb, esem).wait()
        pltpu.make_async_copy(dst_p.at[pl.ds(0, EB)], db, esem).wait()
        pltpu.make_async_copy(w_p.at[pl.ds(0, EB)], wb, esem).wait()

    def mask_and_gather(i, bufs):
        rw, sb, db, _, dl = bufs
        e0 = start + i * EB
        for k in range(EB // L):
            dv = db[pl.ds(k * L, L)]
            ev = e0 + k * L + iot
            ok = (ev < end) & (dv >= dlo) & (dv < dlo + HALF)
            dl[pl.ds(k * L, L)] = jnp.where(ok, dv - dlo, DUMMY)
        hh = EB // 2
        pltpu.async_copy(emb_in.at[sb.at[pl.ds(0, hh)]], rw.at[pl.ds(0, hh)], gsem)
        pltpu.async_copy(emb_in.at[sb.at[pl.ds(hh, hh)]], rw.at[pl.ds(hh, hh)], gsem2)

    def wait_gather(bufs):
        hh = EB // 2
        pltpu.make_async_copy(emb_in.at[bufs[1].at[pl.ds(0, hh)]], bufs[0].at[pl.ds(0, hh)], gsem).wait()
        pltpu.make_async_copy(emb_in.at[bufs[1].at[pl.ds(hh, hh)]], bufs[0].at[pl.ds(hh, hh)], gsem2).wait()

    def scale(bufs):
        rw, _, _, wb, _ = bufs

        @plsc.parallel_loop(0, EB, 4, unroll=2)
        def _(r):
            ws = [plsc.load_gather(wb, [jnp.full((L,), r + u, jnp.int32)])
                  for u in range(4)]
            for q in range(DIM // L):
                for u in range(4):
                    s = pl.ds(q * L, L)
                    rw[r + u, s] = rw[r + u, s] * ws[u]

    def start_scatter(bufs):
        pltpu.async_copy(bufs[0], slab.at[bufs[4]], ssem, add=True)

    def wait_scatter(bufs):
        pltpu.make_async_copy(bufs[0], slab.at[bufs[4]], ssem).wait()

    @pl.when(nblocks > 0)
    def _():
        start_edges(0, BUFS[0])
        wait_edges(BUFS[0])
        mask_and_gather(0, BUFS[0])

    def step(i, cur, nxt):
        @pl.when(i < nblocks)
        def _():
            nok = (i + 1) < nblocks

            @pl.when(nok)
            def _():
                start_edges(i + 1, nxt)

            wait_gather(cur)

            @pl.when(i >= 1)
            def _():
                wait_scatter(nxt)

            @pl.when(nok)
            def _():
                wait_edges(nxt)
                mask_and_gather(i + 1, nxt)

            scale(cur)
            start_scatter(cur)

    def pair(i2, _):
        i = i2 * 2
        step(i, BUFS[0], BUFS[1])
        step(i + 1, BUFS[1], BUFS[0])
        return 0

    lax.fori_loop(0, (nblocks + 1) // 2, pair, 0)

    @pl.when(nblocks % 2 == 1)
    def _():
        wait_scatter(BUFS[0])

    @pl.when((nblocks > 0) & (nblocks % 2 == 0))
    def _():
        wait_scatter(BUFS[1])

    plsc.subcore_barrier()

    gbase = cid * HALF + row0

    def copy_chunk(off, n):
        pltpu.sync_copy(slab.at[pl.ds(row0 + off, n)], obuf.at[pl.ds(0, n)])
        pltpu.sync_copy(obuf.at[pl.ds(0, n)], emb_out.at[pl.ds(gbase + off, n)])

    for k in range(NCHUNK - 1):
        copy_chunk(k * CHUNK, CHUNK)
    tail_off = (NCHUNK - 1) * CHUNK

    @pl.when(sid < NS - 1)
    def _():
        copy_chunk(tail_off, CHUNK)

    @pl.when(sid == NS - 1)
    def _():
        copy_chunk(tail_off, HALF - (NS - 1) * TILE_ROWS - tail_off)


_layer = functools.partial(
    pl.kernel,
    out_type=jax.ShapeDtypeStruct((N_NODES, DIM), jnp.float32),
    mesh=_mesh,
    compiler_params=pltpu.CompilerParams(needs_layout_passes=False, use_tc_tiling_on_sc=False),
    scratch_types=[
        pltpu.VMEM_SHARED((SLAB_ROWS, DIM), jnp.float32),
        pltpu.VMEM((CHUNK, DIM), jnp.float32),
        pltpu.VMEM((EB, DIM), jnp.float32),
        pltpu.VMEM((EB, DIM), jnp.float32),
        pltpu.VMEM((EB,), jnp.int32),
        pltpu.VMEM((EB,), jnp.int32),
        pltpu.VMEM((EB,), jnp.int32),
        pltpu.VMEM((EB,), jnp.int32),
        pltpu.VMEM((EB,), jnp.float32),
        pltpu.VMEM((EB,), jnp.float32),
        pltpu.VMEM((EB,), jnp.int32),
        pltpu.VMEM((EB,), jnp.int32),
        pltpu.VMEM((3 * L,), jnp.int32),
        pltpu.SemaphoreType.DMA,
        pltpu.SemaphoreType.DMA,
        pltpu.SemaphoreType.DMA,
        pltpu.SemaphoreType.DMA,
    ],
)(_layer_body)


def _final_body(uidx_hbm, iidx_hbm, t0, t1, t2, t3, gamma,
                uidxb, iidxb, usum, isum, tmp, obuf1, gsem, hsem):
    cid = lax.axis_index("c")
    sid = lax.axis_index("s")
    iot = lax.iota(jnp.int32, L)
    wid = cid * NS + sid
    pbase = pl.multiple_of(wid * PB, 8)
    pltpu.sync_copy(uidx_hbm.at[pl.ds(pbase, PB)], uidxb)
    pltpu.sync_copy(iidx_hbm.at[pl.ds(pbase, PB)], iidxb)

    gu = pltpu.async_copy(t0.at[uidxb], usum, gsem)
    gi = pltpu.async_copy(t0.at[iidxb], isum, hsem)
    gu.wait()
    gi.wait()

    def accum(dst):
        def addrow(r, _):
            for q in range(DIM // L):
                s = pl.ds(q * L, L)
                dst[r, s] = dst[r, s] + tmp[r, s]
            return 0
        lax.fori_loop(0, PB, addrow, 0)

    for tk in (t1, t2, t3):
        pltpu.async_copy(tk.at[uidxb], tmp, gsem).wait()
        accum(usum)
        pltpu.async_copy(tk.at[iidxb], tmp, hsem).wait()
        accum(isum)

    def grp(g, _):
        ridx = g * L + iot

        def dstep(d, acc):
            colv = jnp.full((L,), d, jnp.int32)
            u = plsc.load_gather(usum, [ridx, colv])
            v = plsc.load_gather(isum, [ridx, colv])
            return acc + u * v

        acc = lax.fori_loop(0, DIM, dstep, jnp.zeros((L,), jnp.float32))
        obuf1[pl.ds(g * L, L)] = acc * (1.0 / 16.0)
        return 0

    lax.fori_loop(0, PB // L, grp, 0)
    pltpu.sync_copy(obuf1, gamma.at[pl.ds(pbase, PB)])


_final = functools.partial(
    pl.kernel,
    out_type=jax.ShapeDtypeStruct((BATCH,), jnp.float32),
    mesh=_mesh,
    compiler_params=pltpu.CompilerParams(needs_layout_passes=False, use_tc_tiling_on_sc=False),
    scratch_types=[
        pltpu.VMEM((PB,), jnp.int32),
        pltpu.VMEM((PB,), jnp.int32),
        pltpu.VMEM((PB, DIM), jnp.float32),
        pltpu.VMEM((PB, DIM), jnp.float32),
        pltpu.VMEM((PB, DIM), jnp.float32),
        pltpu.VMEM((PB,), jnp.float32),
        pltpu.SemaphoreType.DMA,
        pltpu.SemaphoreType.DMA,
    ],
)(_final_body)


def _build_meta(m, n_edges):
    e_total = jnp.int32(n_edges)
    t = jnp.arange(NS, dtype=jnp.int32)
    c0 = ((m + NS - 1) // NS + 7) // 8 * 8
    s0 = t * c0
    e0 = jnp.minimum(s0 + c0, m)
    nb0 = jnp.maximum((e0 - s0 + EB - 1) // EB, 0)
    m8 = (m // 8) * 8
    c1 = ((e_total - m8 + NS - 1) // NS + 7) // 8 * 8
    s1 = m8 + t * c1
    e1 = jnp.minimum(s1 + c1, e_total)
    nb1 = jnp.maximum((e1 - s1 + EB - 1) // EB, 0)
    return jnp.stack([
        jnp.concatenate([s0, nb0, e0]),
        jnp.concatenate([s1, nb1, e1]),
    ])


def kernel(users, items, edge_src, edge_dst, edge_weight, user_emb, item_emb):
    users = users.astype(jnp.int32)
    items = items.astype(jnp.int32) + N_USERS
    edge_src = edge_src.astype(jnp.int32)
    edge_dst = edge_dst.astype(jnp.int32)
    emb0 = jnp.concatenate([user_emb, item_emb], axis=0)
    m = jnp.searchsorted(edge_dst, HALF).astype(jnp.int32)
    meta = _build_meta(m, edge_src.shape[0])
    src_p = jnp.concatenate([edge_src, jnp.zeros((EB,), jnp.int32)])
    dst_p = jnp.concatenate([edge_dst, jnp.full((EB,), N_NODES, jnp.int32)])
    w_p = jnp.concatenate([edge_weight, jnp.zeros((EB,), jnp.float32)])
    e1 = _layer(emb0, src_p, dst_p, w_p, meta)
    e2 = _layer(e1, src_p, dst_p, w_p, meta)
    e3 = _layer(e2, src_p, dst_p, w_p, meta)
    return _final(users, items, emb0, e1, e2, e3)
